# baseline (speedup 1.0000x reference)
import functools

import jax
import jax.numpy as jnp
from jax.experimental import pallas as pl
from jax.experimental.pallas import tpu as pltpu

_N_SYS = 2048
_EPS = 1e-6


def _seg_kernel(sis_off_ref, pemd_off_ref,
                s_m2cx_ref, s_m2cy_ref, s_cc_ref, s_th_ref, s_cx_ref,
                s_cy_ref,
                p_m2cx_ref, p_m2cy_ref, p_cc_ref, p_b0_ref, p_b1_ref,
                p_cx_ref, p_cy_ref,
                g2_ref, gx_ref, gy_ref, out_ref, *, rr, cc, bsys):
    s = pl.program_id(0)
    g2 = g2_ref[...]
    gxp = gx_ref[...]
    gyp = gy_ref[...]

    def sis_pair(lo, hi):
        def body(t, carry):
            a, bx, by = carry
            i0 = lo + 2 * t
            i1 = jnp.minimum(i0 + 1, hi - 1)
            w1 = jnp.where(i0 + 1 < hi, 1.0, 0.0)
            u0 = g2 + s_cc_ref[i0]
            u1 = g2 + s_cc_ref[i1]
            u0 = u0 + s_m2cx_ref[i0] * gxp
            u1 = u1 + s_m2cx_ref[i1] * gxp
            u0 = u0 + s_m2cy_ref[i0] * gyp
            u1 = u1 + s_m2cy_ref[i1] * gyp
            c0 = s_th_ref[i0] * jax.lax.rsqrt(u0)
            c1 = (s_th_ref[i1] * w1) * jax.lax.rsqrt(u1)
            a = a + c0 + c1
            bx = bx + s_cx_ref[i0] * c0 + s_cx_ref[i1] * c1
            by = by + s_cy_ref[i0] * c0 + s_cy_ref[i1] * c1
            return a, bx, by
        return body

    def pemd_pair(lo, hi):
        def body(t, carry):
            a, bx, by = carry
            i0 = lo + 2 * t
            i1 = jnp.minimum(i0 + 1, hi - 1)
            wlog = jnp.where(i0 + 1 < hi, 0.0, -jnp.inf)
            u0 = g2 + p_cc_ref[i0]
            u1 = g2 + p_cc_ref[i1]
            u0 = u0 + p_m2cx_ref[i0] * gxp
            u1 = u1 + p_m2cx_ref[i1] * gxp
            u0 = u0 + p_m2cy_ref[i0] * gyp
            u1 = u1 + p_m2cy_ref[i1] * gyp
            c0 = jnp.exp(p_b0_ref[i0] + p_b1_ref[i0] * jnp.log(u0))
            c1 = jnp.exp((p_b0_ref[i1] + wlog) + p_b1_ref[i1] * jnp.log(u1))
            c0 = c0 * jax.lax.rsqrt(u0)
            c1 = c1 * jax.lax.rsqrt(u1)
            a = a + c0 + c1
            bx = bx + p_cx_ref[i0] * c0 + p_cx_ref[i1] * c1
            by = by + p_cy_ref[i0] * c0 + p_cy_ref[i1] * c1
            return a, bx, by
        return body

    zero = jnp.zeros((rr, cc), jnp.float32)
    for j in range(bsys):
        sysid = s * bsys + j
        slo = sis_off_ref[sysid]
        shi = sis_off_ref[sysid + 1]
        plo = pemd_off_ref[sysid]
        phi = pemd_off_ref[sysid + 1]
        carry = jax.lax.fori_loop(0, (shi - slo + 1) // 2,
                                  sis_pair(slo, shi), (zero, zero, zero))
        a, bx, by = jax.lax.fori_loop(0, (phi - plo + 1) // 2,
                                      pemd_pair(plo, phi), carry)
        na = 1.0 - a
        out_ref[j, 0] = gxp * na + bx
        out_ref[j, 1] = gyp * na + by


def _offsets(idx):
    counts = jnp.bincount(idx, length=_N_SYS)
    return jnp.concatenate(
        [jnp.zeros((1,), jnp.int32),
         jnp.cumsum(counts).astype(jnp.int32)])


@jax.jit
def kernel(lens_grid, sis_params, pemd_params, sis_idx, pemd_idx):
    hh, ww, _ = lens_grid.shape
    rr = hh * ww // 128
    gx = lens_grid[:, :, 0].reshape(rr, 128)
    gy = lens_grid[:, :, 1].reshape(rr, 128)
    g2 = gx * gx + gy * gy + _EPS

    so = jnp.argsort(sis_idx)
    sp = sis_params[so]
    s_th, s_cx, s_cy = sp[:, 0], sp[:, 1], sp[:, 2]
    s_m2cx = -2.0 * s_cx
    s_m2cy = -2.0 * s_cy
    s_cc = s_cx * s_cx + s_cy * s_cy
    sis_off = _offsets(sis_idx)

    po = jnp.argsort(pemd_idx)
    pp = pemd_params[po]
    th, gam, p_cx, p_cy = pp[:, 0], pp[:, 1], pp[:, 2], pp[:, 3]
    p_b0 = (gam - 1.0) * jnp.log(th)
    p_b1 = 0.5 * (2.0 - gam)
    p_m2cx = -2.0 * p_cx
    p_m2cy = -2.0 * p_cy
    p_cc = p_cx * p_cx + p_cy * p_cy
    pemd_off = _offsets(pemd_idx)

    bsys = 16
    out = pl.pallas_call(
        functools.partial(_seg_kernel, rr=rr, cc=128, bsys=bsys),
        grid=(_N_SYS // bsys,),
        in_specs=[pl.BlockSpec(memory_space=pltpu.SMEM)] * 15 + [
            pl.BlockSpec((rr, 128), lambda s: (0, 0)),
            pl.BlockSpec((rr, 128), lambda s: (0, 0)),
            pl.BlockSpec((rr, 128), lambda s: (0, 0)),
        ],
        out_specs=pl.BlockSpec((bsys, 2, rr, 128), lambda s: (s, 0, 0, 0)),
        out_shape=jax.ShapeDtypeStruct((_N_SYS, 2, rr, 128), jnp.float32),
    )(sis_off, pemd_off,
      s_m2cx, s_m2cy, s_cc, s_th, s_cx, s_cy,
      p_m2cx, p_m2cy, p_cc, p_b0, p_b1, p_cx, p_cy,
      g2, gx, gy)
    return out.reshape(_N_SYS, 2, hh, ww).transpose(0, 2, 3, 1)


# speedup vs baseline: 1.0035x; 1.0035x over previous
import functools

import jax
import jax.numpy as jnp
from jax.experimental import pallas as pl
from jax.experimental.pallas import tpu as pltpu

_N_SYS = 2048
_EPS = 1e-6


def _seg_kernel(off_ref, m2cx_ref, m2cy_ref, cc_ref, b0_ref, b1_ref,
                cx_ref, cy_ref, g2_ref, gx_ref, gy_ref, out_ref,
                *, rr, cc, bsys):
    s = pl.program_id(0)
    g2 = g2_ref[...]
    gxp = gx_ref[...]
    gyp = gy_ref[...]

    def body(i, carry):
        a, bx, by = carry
        u = g2 + cc_ref[i]
        u = u + m2cx_ref[i] * gxp
        u = u + m2cy_ref[i] * gyp
        coef = jnp.exp(b0_ref[i] + b1_ref[i] * jnp.log(u))
        coef = coef * jax.lax.rsqrt(u)
        return a + coef, bx + cx_ref[i] * coef, by + cy_ref[i] * coef

    zero = jnp.zeros((rr, cc), jnp.float32)
    for j in range(bsys):
        sysid = s * bsys + j
        a, bx, by = jax.lax.fori_loop(off_ref[sysid], off_ref[sysid + 1],
                                      body, (zero, zero, zero))
        na = 1.0 - a
        out_ref[j, 0] = gxp * na + bx
        out_ref[j, 1] = gyp * na + by


@jax.jit
def kernel(lens_grid, sis_params, pemd_params, sis_idx, pemd_idx):
    hh, ww, _ = lens_grid.shape
    rr = hh * ww // 128
    gx = lens_grid[:, :, 0].reshape(rr, 128)
    gy = lens_grid[:, :, 1].reshape(rr, 128)
    g2 = gx * gx + gy * gy + _EPS

    th = jnp.concatenate([sis_params[:, 0], pemd_params[:, 0]])
    gam = jnp.concatenate([jnp.full(sis_params.shape[:1], 2.0),
                           pemd_params[:, 1]])
    cx = jnp.concatenate([sis_params[:, 1], pemd_params[:, 2]])
    cy = jnp.concatenate([sis_params[:, 2], pemd_params[:, 3]])
    idx = jnp.concatenate([sis_idx, pemd_idx])

    order = jnp.argsort(idx)
    th = th[order]
    gam = gam[order]
    cx = cx[order]
    cy = cy[order]

    b0 = (gam - 1.0) * jnp.log(th)
    b1 = 0.5 * (2.0 - gam)
    m2cx = -2.0 * cx
    m2cy = -2.0 * cy
    ccs = cx * cx + cy * cy
    counts = jnp.bincount(idx, length=_N_SYS)
    off = jnp.concatenate(
        [jnp.zeros((1,), jnp.int32),
         jnp.cumsum(counts).astype(jnp.int32)])

    bsys = 16
    out = pl.pallas_call(
        functools.partial(_seg_kernel, rr=rr, cc=128, bsys=bsys),
        grid=(_N_SYS // bsys,),
        in_specs=[pl.BlockSpec(memory_space=pltpu.SMEM)] * 8 + [
            pl.BlockSpec((rr, 128), lambda s: (0, 0)),
            pl.BlockSpec((rr, 128), lambda s: (0, 0)),
            pl.BlockSpec((rr, 128), lambda s: (0, 0)),
        ],
        out_specs=pl.BlockSpec((bsys, 2, rr, 128), lambda s: (s, 0, 0, 0)),
        out_shape=jax.ShapeDtypeStruct((_N_SYS, 2, rr, 128), jnp.float32),
    )(off, m2cx, m2cy, ccs, b0, b1, cx, cy, g2, gx, gy)
    return out.reshape(_N_SYS, 2, hh, ww).transpose(0, 2, 3, 1)
